# baseline (device time: 65357 ns/iter reference)
import jax
import jax.numpy as jnp
from jax import lax
from jax.experimental import pallas as pl
from jax.experimental.pallas import tpu as pltpu

Z = 4
AW = 128


def kernel(x, assign, W1, W2):
    T, D = x.shape
    E, _, F = W1.shape
    CW = D + AW

    xb = x.astype(jnp.bfloat16)
    ab = assign.reshape(T, 1).astype(jnp.bfloat16)
    w1b = W1.astype(jnp.bfloat16)
    w2b = W2.astype(jnp.bfloat16)

    def body(x_ref, a_ref, w1_ref, w2_ref, out_ref,
             comm, pbuf, rbuf, ag_send, ag_recv, rs_send, rs_recv):
        mx = lax.axis_index("x")
        my = lax.axis_index("y")
        mz = lax.axis_index("z")
        right = lax.rem(mz + 1, Z)

        barrier = pltpu.get_barrier_semaphore()
        for d in range(1, Z):
            peer = lax.rem(mz + d, Z)
            pl.semaphore_signal(
                barrier, inc=1,
                device_id=(mx, my, peer),
                device_id_type=pl.DeviceIdType.MESH,
            )
        pl.semaphore_wait(barrier, Z - 1)

        comm[0, :, :D] = x_ref[...]
        comm[0, :, D:] = jnp.broadcast_to(a_ref[...], (T, AW))

        for h in range(Z - 1):
            rdma = pltpu.make_async_remote_copy(
                src_ref=comm.at[(Z - h) % Z],
                dst_ref=comm.at[Z - 1 - h],
                send_sem=ag_send.at[h],
                recv_sem=ag_recv.at[h],
                device_id=(mx, my, right),
                device_id_type=pl.DeviceIdType.MESH,
            )
            rdma.start()
            rdma.wait()

        w1s = [w1_ref[e] for e in range(E)]
        w2s = [w2_ref[e] for e in range(E)]
        for d in range(Z):
            blk = comm[d]
            xd = blk[:, :D]
            av = blk[:, D:D + 1]
            acc = jnp.zeros((T, D), jnp.float32)
            for k in range(E):
                eb = (E * mz + k).astype(jnp.bfloat16)
                xm = jnp.where(av == eb, xd, jnp.bfloat16(0))
                h1 = lax.dot(xm, w1s[k], preferred_element_type=jnp.float32)
                hk = jnp.maximum(h1, 0.0).astype(jnp.bfloat16)
                acc = acc + lax.dot(hk, w2s[k], preferred_element_type=jnp.float32)
            pbuf[d] = acc.astype(jnp.bfloat16)

        rdmas = []
        for d in range(1, Z):
            tz = lax.rem(mz + d, Z)
            r = pltpu.make_async_remote_copy(
                src_ref=pbuf.at[d],
                dst_ref=rbuf.at[d - 1],
                send_sem=rs_send.at[d - 1],
                recv_sem=rs_recv.at[d - 1],
                device_id=(mx, my, tz),
                device_id_type=pl.DeviceIdType.MESH,
            )
            r.start()
            rdmas.append(r)
        for r in rdmas:
            r.wait()

        acc = pbuf[0].astype(jnp.float32)
        for j in range(Z - 1):
            acc = acc + rbuf[j].astype(jnp.float32)
        out_ref[...] = acc

    return pl.pallas_call(
        body,
        out_shape=jax.ShapeDtypeStruct((T, D), jnp.float32),
        in_specs=[pl.BlockSpec(memory_space=pltpu.VMEM)] * 4,
        out_specs=pl.BlockSpec(memory_space=pltpu.VMEM),
        scratch_shapes=[
            pltpu.VMEM((Z, T, CW), jnp.bfloat16),
            pltpu.VMEM((Z, T, D), jnp.bfloat16),
            pltpu.VMEM((Z - 1, T, D), jnp.bfloat16),
            pltpu.SemaphoreType.DMA((Z - 1,)),
            pltpu.SemaphoreType.DMA((Z - 1,)),
            pltpu.SemaphoreType.DMA((Z - 1,)),
            pltpu.SemaphoreType.DMA((Z - 1,)),
        ],
        compiler_params=pltpu.CompilerParams(collective_id=0),
    )(xb, ab, w1b, w2b)


# device time: 56114 ns/iter; 1.1647x vs baseline; 1.1647x over previous
import jax
import jax.numpy as jnp
from jax import lax
from jax.experimental import pallas as pl
from jax.experimental.pallas import tpu as pltpu

Z = 4
AW = 128


def kernel(x, assign, W1, W2):
    T, D = x.shape
    E, _, F = W1.shape
    CW = D + AW

    xb = x.astype(jnp.bfloat16)
    ab = assign.reshape(T, 1).astype(jnp.bfloat16)
    w1b = W1.astype(jnp.bfloat16)
    w2b = W2.astype(jnp.bfloat16)

    def body(x_ref, a_ref, w1_ref, w2_ref, out_ref,
             comm, pbuf, rbuf, ag_send, ag_recv, rs_send, rs_recv):
        mx = lax.axis_index("x")
        my = lax.axis_index("y")
        mz = lax.axis_index("z")
        right = lax.rem(mz + 1, Z)
        left = lax.rem(mz + Z - 1, Z)

        barrier = pltpu.get_barrier_semaphore()
        for d in range(1, Z):
            peer = lax.rem(mz + d, Z)
            pl.semaphore_signal(
                barrier, inc=1,
                device_id=(mx, my, peer),
                device_id_type=pl.DeviceIdType.MESH,
            )
        pl.semaphore_wait(barrier, Z - 1)

        comm[0, :, :D] = x_ref[...]
        comm[0, :, D:] = jnp.broadcast_to(a_ref[...], (T, AW))

        ag_a = pltpu.make_async_remote_copy(
            src_ref=comm.at[0], dst_ref=comm.at[Z - 1],
            send_sem=ag_send.at[0], recv_sem=ag_recv.at[0],
            device_id=(mx, my, right), device_id_type=pl.DeviceIdType.MESH,
        )
        ag_b = pltpu.make_async_remote_copy(
            src_ref=comm.at[0], dst_ref=comm.at[1],
            send_sem=ag_send.at[1], recv_sem=ag_recv.at[1],
            device_id=(mx, my, left), device_id_type=pl.DeviceIdType.MESH,
        )
        ag_c = pltpu.make_async_remote_copy(
            src_ref=comm.at[Z - 1], dst_ref=comm.at[Z - 2],
            send_sem=ag_send.at[2], recv_sem=ag_recv.at[2],
            device_id=(mx, my, right), device_id_type=pl.DeviceIdType.MESH,
        )

        w1s = [w1_ref[e] for e in range(E)]
        w2s = [w2_ref[e] for e in range(E)]

        def expert_block(d):
            blk = comm[d]
            xd = blk[:, :D]
            av = blk[:, D:D + 1]
            acc = jnp.zeros((T, D), jnp.float32)
            for k in range(E):
                eb = (E * mz + k).astype(jnp.bfloat16)
                xm = jnp.where(av == eb, xd, jnp.bfloat16(0))
                h1 = lax.dot(xm, w1s[k], preferred_element_type=jnp.float32)
                hk = jnp.maximum(h1, 0.0).astype(jnp.bfloat16)
                acc = acc + lax.dot(hk, w2s[k], preferred_element_type=jnp.float32)
            pbuf[d] = acc.astype(jnp.bfloat16)

        def rs_start(d):
            tz = lax.rem(mz + d, Z)
            r = pltpu.make_async_remote_copy(
                src_ref=pbuf.at[d],
                dst_ref=rbuf.at[d - 1],
                send_sem=rs_send.at[d - 1],
                recv_sem=rs_recv.at[d - 1],
                device_id=(mx, my, tz),
                device_id_type=pl.DeviceIdType.MESH,
            )
            r.start()
            return r

        ag_a.start()
        ag_b.start()
        expert_block(0)
        ag_a.wait_recv()
        ag_c.start()
        expert_block(Z - 1)
        rs = {Z - 1: rs_start(Z - 1)}
        ag_b.wait_recv()
        expert_block(1)
        rs[1] = rs_start(1)
        ag_c.wait_recv()
        expert_block(Z - 2)
        rs[Z - 2] = rs_start(Z - 2)

        for r in (ag_a, ag_b, ag_c):
            r.wait_send()
        for d in range(1, Z):
            rs[d].wait_recv()
        for d in range(1, Z):
            rs[d].wait_send()

        acc = pbuf[0].astype(jnp.float32)
        for j in range(Z - 1):
            acc = acc + rbuf[j].astype(jnp.float32)
        out_ref[...] = acc

    return pl.pallas_call(
        body,
        out_shape=jax.ShapeDtypeStruct((T, D), jnp.float32),
        in_specs=[pl.BlockSpec(memory_space=pltpu.VMEM)] * 4,
        out_specs=pl.BlockSpec(memory_space=pltpu.VMEM),
        scratch_shapes=[
            pltpu.VMEM((Z, T, CW), jnp.bfloat16),
            pltpu.VMEM((Z, T, D), jnp.bfloat16),
            pltpu.VMEM((Z - 1, T, D), jnp.bfloat16),
            pltpu.SemaphoreType.DMA((Z - 1,)),
            pltpu.SemaphoreType.DMA((Z - 1,)),
            pltpu.SemaphoreType.DMA((Z - 1,)),
            pltpu.SemaphoreType.DMA((Z - 1,)),
        ],
        compiler_params=pltpu.CompilerParams(collective_id=0),
    )(xb, ab, w1b, w2b)
